# baseline (speedup 1.0000x reference)
import functools

import jax
import jax.numpy as jnp
from jax import lax
from jax.experimental import pallas as pl
from jax.experimental.pallas import tpu as pltpu
from jax.experimental.pallas import tpu_sc as plsc

N_NODES = 10000
N_EDGES = 320000
C = 128
NB = 32
N_SPECIES = 100
G = 64
CUTOFF = 5.0
GAMMA = (NB / CUTOFF) ** 2

NCORE = 2
NSUB = 16
NW = NCORE * NSUB
NPAD = 10240
ROWS_PER_W = NPAD // NW
E_PER_W = N_EDGES // NW
IW = 80
MIW = 40
MJROWS = E_PER_W // MIW
MJPAD = 256
MJSTRIDE = MJPAD + 32
TRASH = NPAD - 1
ECH = 2000
EB = 512
NBLK = 256

_mesh = plsc.VectorSubcoreMesh(
    core_axis_name="c", subcore_axis_name="s",
    num_cores=NCORE, num_subcores=NSUB)
_sc_params = pltpu.CompilerParams(needs_layout_passes=False)


@functools.partial(
    pl.kernel,
    out_type=[jax.ShapeDtypeStruct((NPAD, C), jnp.float32),
              jax.ShapeDtypeStruct((N_EDGES,), jnp.float32)],
    mesh=_mesh,
    compiler_params=_sc_params,
    scratch_types=[
        pltpu.VMEM((NPAD,), jnp.float32),
        pltpu.VMEM((NPAD,), jnp.float32),
        pltpu.VMEM((NPAD,), jnp.float32),
        pltpu.VMEM((ECH,), jnp.int32),
        pltpu.VMEM((ECH,), jnp.int32),
        pltpu.VMEM((ECH,), jnp.float32),
        pltpu.VMEM((4, IW), jnp.int32),
        pltpu.VMEM((IW, C), jnp.float32),
        pltpu.SemaphoreType.DMA,
    ],
)
def _sc_prep(an2d_h, cx_h, cy_h, cz_h, src_h, dst_h, embed_h, x0_h, sq_h,
             cxv, cyv, czv, sidx, didx, sqv, anv, xrows, sem):
    cid = lax.axis_index("c")
    sid = lax.axis_index("s")
    wid = cid * NSUB + sid

    pltpu.sync_copy(an2d_h.at[pl.ds(wid * 4, 4)], anv)
    for j in range(4):
        pltpu.async_copy(embed_h.at[anv.at[j]], xrows, sem).wait()
        pltpu.sync_copy(xrows, x0_h.at[pl.ds(wid * ROWS_PER_W + j * IW, IW)])

    pltpu.sync_copy(cx_h, cxv)
    pltpu.sync_copy(cy_h, cyv)
    pltpu.sync_copy(cz_h, czv)

    ebase = wid * E_PER_W

    def chunk_body(k, _):
        base = ebase + k * ECH
        pltpu.sync_copy(src_h.at[pl.ds(base, ECH)], sidx)
        pltpu.sync_copy(dst_h.at[pl.ds(base, ECH)], didx)

        def g_body(g, _):
            sl = pl.ds(g * 16, 16)
            s16 = sidx[sl]
            d16 = didx[sl]
            dx = plsc.load_gather(cxv, [d16]) - plsc.load_gather(cxv, [s16])
            dy = plsc.load_gather(cyv, [d16]) - plsc.load_gather(cyv, [s16])
            dz = plsc.load_gather(czv, [d16]) - plsc.load_gather(czv, [s16])
            sqv[sl] = dx * dx + dy * dy + dz * dz
            return 0

        lax.fori_loop(0, ECH // 16, g_body, 0)
        pltpu.sync_copy(sqv, sq_h.at[pl.ds(base, ECH)])
        return 0

    lax.fori_loop(0, E_PER_W // ECH, chunk_body, 0)


@functools.partial(
    pl.kernel,
    out_type=jax.ShapeDtypeStruct((NCORE * NPAD, C), jnp.float32),
    mesh=_mesh,
    compiler_params=_sc_params,
    scratch_types=[
        pltpu.VMEM((2, 32, MIW), jnp.int32),
        pltpu.VMEM((2, 32, MIW), jnp.int32),
        pltpu.VMEM((4, MIW, C), jnp.float32),
        pltpu.VMEM((2, MIW, C), jnp.float32),
        pltpu.VMEM_SHARED((NPAD, C), jnp.float32),
        pltpu.SemaphoreType.DMA, pltpu.SemaphoreType.DMA,
        pltpu.SemaphoreType.DMA, pltpu.SemaphoreType.DMA,
        pltpu.SemaphoreType.DMA, pltpu.SemaphoreType.DMA,
        pltpu.SemaphoreType.DMA, pltpu.SemaphoreType.DMA,
        pltpu.SemaphoreType.DMA, pltpu.SemaphoreType.DMA,
    ],
)
def _sc_msgpass(x_h, filt_h, src3_h, dst3_h, parts_h,
                sidx2, didx2, rows4, fv2, acc,
                g0, g1, g2, g3, f0, f1, s0, s1, s2, s3):
    cid = lax.axis_index("c")
    sid = lax.axis_index("s")
    wid = cid * NSUB + sid
    zrows = NPAD // NSUB
    gsem = (g0, g1, g2, g3)
    fsem = (f0, f1)
    ssem = (s0, s1, s2, s3)
    ebase = wid * E_PER_W
    ibase = wid * MJSTRIDE

    def filt_off(j):
        return jnp.where(j < MJROWS, ebase + j * MIW, ebase)

    def wait_g(b):
        pltpu.make_async_copy(x_h.at[pl.ds(0, MIW)], rows4.at[b],
                              gsem[b]).wait()

    def wait_f(b):
        pltpu.make_async_copy(filt_h.at[pl.ds(0, MIW)], fv2.at[b],
                              fsem[b]).wait()

    def wait_s(b):
        pltpu.make_async_copy(x_h.at[pl.ds(0, MIW)], rows4.at[b],
                              ssem[b]).wait()

    pltpu.sync_copy(src3_h.at[pl.ds(ibase, 32)], sidx2.at[0])
    pltpu.sync_copy(dst3_h.at[pl.ds(ibase, 32)], didx2.at[0])
    zbuf = rows4.at[3]

    def zr(r, _):
        for c8 in range(C // 16):
            zbuf[r, pl.ds(c8 * 16, 16)] = jnp.zeros((16,), jnp.float32)
        return 0
    lax.fori_loop(0, MIW, zr, 0)

    def zc(i, _):
        pltpu.sync_copy(zbuf, acc.at[pl.ds(sid * zrows + i * MIW, MIW)])
        return 0
    lax.fori_loop(0, zrows // MIW, zc, 0)
    plsc.subcore_barrier()

    pltpu.async_copy(zbuf, acc.at[pl.ds(NPAD - MIW, MIW)], ssem[3])
    for b in range(3):
        pltpu.async_copy(x_h.at[sidx2.at[0, b]], rows4.at[b], gsem[b])
    for b in range(2):
        pltpu.async_copy(filt_h.at[pl.ds(filt_off(b), MIW)],
                         fv2.at[b], fsem[b])

    def chunk_body(c, _):
        cb = lax.rem(c, 2)

        def quad_body(q, _):
            for k in range(4):
                j = c * 32 + q * 4 + k
                bg = (k + 3) % 4
                jg = j + 3
                cbg = lax.rem(jg // 32, 2)
                rowg = lax.rem(jg, 32)
                jf = j + 2

                wait_s(bg)
                pltpu.async_copy(x_h.at[sidx2.at[cbg, rowg]],
                                 rows4.at[bg], gsem[bg])
                wait_g(k)
                wait_f(k % 2)

                def mb(r, _):
                    for u in range(2):
                        for c8 in range(C // 16):
                            sl = pl.ds(c8 * 16, 16)
                            rows4[k, r * 2 + u, sl] = (
                                rows4[k, r * 2 + u, sl]
                                * fv2[k % 2, r * 2 + u, sl])
                    return 0
                lax.fori_loop(0, MIW // 2, mb, 0)

                pltpu.async_copy(rows4.at[k], acc.at[didx2.at[cb, q * 4 + k]],
                                 ssem[k], add=True)
                pltpu.async_copy(filt_h.at[pl.ds(filt_off(jf), MIW)],
                                 fv2.at[k % 2], fsem[k % 2])
            return 0

        quad_body(0, 0)
        cb1 = lax.rem(c + 1, 2)
        pltpu.sync_copy(src3_h.at[pl.ds(ibase + (c + 1) * 32, 32)],
                        sidx2.at[cb1])
        pltpu.sync_copy(dst3_h.at[pl.ds(ibase + (c + 1) * 32, 32)],
                        didx2.at[cb1])
        lax.fori_loop(1, 8, quad_body, 0)
        return 0

    lax.fori_loop(0, MJPAD // 32, chunk_body, 0)

    for b in range(3):
        wait_g(b)
    for b in range(2):
        wait_f(b)
    wait_s(3)
    plsc.subcore_barrier()

    pltpu.sync_copy(acc.at[pl.ds(sid * zrows, zrows)],
                    parts_h.at[pl.ds(cid * NPAD + sid * zrows, zrows)])


def _sigmoid(z):
    return 1.0 / (1.0 + jnp.exp(-z))


def _filter_body(sq_ref, w_ref, b_ref, f_ref):
    d = jnp.sqrt(sq_ref[:] + 1e-8)
    cent = lax.broadcasted_iota(jnp.int32, (1, NB), 1).astype(jnp.float32)
    cent = cent * (CUTOFF / (NB - 1))
    diff = d - cent
    rbf = jnp.exp(-GAMMA * diff * diff)
    z = jnp.dot(rbf, w_ref[:], preferred_element_type=jnp.float32) + b_ref[:]
    f_ref[:] = z * _sigmoid(z)


def _tc_filter(sq2, w, b):
    grid = N_EDGES // EB
    return pl.pallas_call(
        _filter_body,
        grid=(grid,),
        in_specs=[
            pl.BlockSpec((EB, 1), lambda i: (i, 0)),
            pl.BlockSpec((NB, C), lambda i: (0, 0)),
            pl.BlockSpec((1, C), lambda i: (0, 0)),
        ],
        out_specs=pl.BlockSpec((EB, C), lambda i: (i, 0)),
        out_shape=jax.ShapeDtypeStruct((N_EDGES, C), jnp.float32),
    )(sq2, w, b)


def _update_body(x_ref, p_ref, w_ref, b_ref, o_ref):
    h = x_ref[:] + p_ref[0] + p_ref[1]
    z = jnp.dot(h, w_ref[:], preferred_element_type=jnp.float32) + b_ref[:]
    o_ref[:] = z * _sigmoid(z)


def _tc_update(x, parts, w, b):
    grid = NPAD // NBLK
    return pl.pallas_call(
        _update_body,
        grid=(grid,),
        in_specs=[
            pl.BlockSpec((NBLK, C), lambda i: (i, 0)),
            pl.BlockSpec((NCORE, NBLK, C), lambda i: (0, i, 0)),
            pl.BlockSpec((C, C), lambda i: (0, 0)),
            pl.BlockSpec((1, C), lambda i: (0, 0)),
        ],
        out_specs=pl.BlockSpec((NBLK, C), lambda i: (i, 0)),
        out_shape=jax.ShapeDtypeStruct((NPAD, C), jnp.float32),
    )(x, parts, w, b)


def _final_body(x_ref, p_ref, w_ref, b_ref, bat_ref, wo_ref, bo_ref,
                gx_ref, gc_ref, go_ref):
    i = pl.program_id(0)
    h = x_ref[:] + p_ref[0] + p_ref[1]
    z = jnp.dot(h, w_ref[:], preferred_element_type=jnp.float32) + b_ref[:]
    x2 = z * _sigmoid(z)
    gids = lax.broadcasted_iota(jnp.int32, (1, G), 1)
    onehot = (bat_ref[:] == gids).astype(jnp.float32)
    dn = (((0,), (0,)), ((), ()))
    gpart = lax.dot_general(onehot, x2, dn, preferred_element_type=jnp.float32)
    cpart = lax.dot_general(onehot, jnp.ones_like(x2), dn,
                            preferred_element_type=jnp.float32)

    @pl.when(i == 0)
    def _():
        gx_ref[:] = gpart
        gc_ref[:] = cpart

    @pl.when(i > 0)
    def _():
        gx_ref[:] = gx_ref[:] + gpart
        gc_ref[:] = gc_ref[:] + cpart

    @pl.when(i == pl.num_programs(0) - 1)
    def _():
        go_ref[:] = (jnp.dot(gx_ref[:], wo_ref[:],
                             preferred_element_type=jnp.float32)
                     + gc_ref[:, 0:1] * bo_ref[0, 0])


def _tc_final(x, parts, w, b, bat2, wo, bo):
    grid = NPAD // NBLK
    _, _, go = pl.pallas_call(
        _final_body,
        grid=(grid,),
        in_specs=[
            pl.BlockSpec((NBLK, C), lambda i: (i, 0)),
            pl.BlockSpec((NCORE, NBLK, C), lambda i: (0, i, 0)),
            pl.BlockSpec((C, C), lambda i: (0, 0)),
            pl.BlockSpec((1, C), lambda i: (0, 0)),
            pl.BlockSpec((NBLK, 1), lambda i: (i, 0)),
            pl.BlockSpec((C, 1), lambda i: (0, 0)),
            pl.BlockSpec((1, 1), lambda i: (0, 0)),
        ],
        out_specs=[
            pl.BlockSpec((G, C), lambda i: (0, 0)),
            pl.BlockSpec((G, C), lambda i: (0, 0)),
            pl.BlockSpec((G, 1), lambda i: (0, 0)),
        ],
        out_shape=[
            jax.ShapeDtypeStruct((G, C), jnp.float32),
            jax.ShapeDtypeStruct((G, C), jnp.float32),
            jax.ShapeDtypeStruct((G, 1), jnp.float32),
        ],
        compiler_params=pltpu.CompilerParams(
            dimension_semantics=("arbitrary",)),
    )(x, parts, w, b, bat2, wo, bo)
    return go


def kernel(atomic_number, coordinate, edge_index, batch, embed_table,
           W_rbf0, b_rbf0, W_up0, b_up0,
           W_rbf1, b_rbf1, W_up1, b_up1,
           W_out, b_out):
    an = jnp.pad(atomic_number.astype(jnp.int32), (0, NPAD - N_NODES))
    an2d = an.reshape(NPAD // IW, IW)
    coord_t = jnp.pad(coordinate.T.astype(jnp.float32),
                      ((0, 0), (0, NPAD - N_NODES)))
    cx, cy, cz = coord_t[0], coord_t[1], coord_t[2]
    src = edge_index[0].astype(jnp.int32)
    dst = edge_index[1].astype(jnp.int32)
    src3 = jnp.pad(src.reshape(NW, MJROWS, MIW),
                   ((0, 0), (0, MJSTRIDE - MJROWS), (0, 0))
                   ).reshape(NW * MJSTRIDE, MIW)
    dst3 = jnp.pad(dst.reshape(NW, MJROWS, MIW),
                   ((0, 0), (0, MJSTRIDE - MJROWS), (0, 0)),
                   constant_values=TRASH).reshape(NW * MJSTRIDE, MIW)
    bat2 = jnp.pad(batch.astype(jnp.int32), (0, NPAD - N_NODES),
                   constant_values=G).reshape(NPAD, 1)

    x0, sq = _sc_prep(an2d, cx, cy, cz, src, dst, embed_table)
    sq2 = sq.reshape(N_EDGES, 1)
    filt0 = _tc_filter(sq2, W_rbf0, b_rbf0.reshape(1, C))
    filt1 = _tc_filter(sq2, W_rbf1, b_rbf1.reshape(1, C))
    parts0 = _sc_msgpass(x0, filt0, src3, dst3).reshape(NCORE, NPAD, C)
    x1 = _tc_update(x0, parts0, W_up0, b_up0.reshape(1, C))
    parts1 = _sc_msgpass(x1, filt1, src3, dst3).reshape(NCORE, NPAD, C)
    go = _tc_final(x1, parts1, W_up1, b_up1.reshape(1, C),
                   bat2, W_out, b_out.reshape(1, 1))
    return go


# speedup vs baseline: 1.0113x; 1.0113x over previous
import functools

import jax
import jax.numpy as jnp
from jax import lax
from jax.experimental import pallas as pl
from jax.experimental.pallas import tpu as pltpu
from jax.experimental.pallas import tpu_sc as plsc

N_NODES = 10000
N_EDGES = 320000
C = 128
NB = 32
N_SPECIES = 100
G = 64
CUTOFF = 5.0
GAMMA = (NB / CUTOFF) ** 2

NCORE = 2
NSUB = 16
NW = NCORE * NSUB
NPAD = 10240
ROWS_PER_W = NPAD // NW
E_PER_W = N_EDGES // NW
IW = 80
SIW = 128
E_TILE = 10240
SJROWS = E_TILE // SIW
SCH = 8
TRASH = NPAD - 1
ECH = 2000
EB = 512
NBLK = 256

_mesh = plsc.VectorSubcoreMesh(
    core_axis_name="c", subcore_axis_name="s",
    num_cores=NCORE, num_subcores=NSUB)
_sc_params = pltpu.CompilerParams(needs_layout_passes=False)


@functools.partial(
    pl.kernel,
    out_type=[jax.ShapeDtypeStruct((NPAD, C), jnp.float32),
              jax.ShapeDtypeStruct((N_EDGES,), jnp.float32)],
    mesh=_mesh,
    compiler_params=_sc_params,
    scratch_types=[
        pltpu.VMEM((NPAD,), jnp.float32),
        pltpu.VMEM((NPAD,), jnp.float32),
        pltpu.VMEM((NPAD,), jnp.float32),
        pltpu.VMEM((ECH,), jnp.int32),
        pltpu.VMEM((ECH,), jnp.int32),
        pltpu.VMEM((ECH,), jnp.float32),
        pltpu.VMEM((4, IW), jnp.int32),
        pltpu.VMEM((IW, C), jnp.float32),
        pltpu.SemaphoreType.DMA,
    ],
)
def _sc_prep(an2d_h, cx_h, cy_h, cz_h, src_h, dst_h, embed_h, x0_h, sq_h,
             cxv, cyv, czv, sidx, didx, sqv, anv, xrows, sem):
    cid = lax.axis_index("c")
    sid = lax.axis_index("s")
    wid = cid * NSUB + sid

    pltpu.sync_copy(an2d_h.at[pl.ds(wid * 4, 4)], anv)
    for j in range(4):
        pltpu.async_copy(embed_h.at[anv.at[j]], xrows, sem).wait()
        pltpu.sync_copy(xrows, x0_h.at[pl.ds(wid * ROWS_PER_W + j * IW, IW)])

    pltpu.sync_copy(cx_h, cxv)
    pltpu.sync_copy(cy_h, cyv)
    pltpu.sync_copy(cz_h, czv)

    ebase = wid * E_PER_W

    def chunk_body(k, _):
        base = ebase + k * ECH
        pltpu.sync_copy(src_h.at[pl.ds(base, ECH)], sidx)
        pltpu.sync_copy(dst_h.at[pl.ds(base, ECH)], didx)

        def g_body(g, _):
            sl = pl.ds(g * 16, 16)
            s16 = sidx[sl]
            d16 = didx[sl]
            dx = plsc.load_gather(cxv, [d16]) - plsc.load_gather(cxv, [s16])
            dy = plsc.load_gather(cyv, [d16]) - plsc.load_gather(cyv, [s16])
            dz = plsc.load_gather(czv, [d16]) - plsc.load_gather(czv, [s16])
            sqv[sl] = dx * dx + dy * dy + dz * dz
            return 0

        lax.fori_loop(0, ECH // 16, g_body, 0)
        pltpu.sync_copy(sqv, sq_h.at[pl.ds(base, ECH)])
        return 0

    lax.fori_loop(0, E_PER_W // ECH, chunk_body, 0)


@functools.partial(
    pl.kernel,
    out_type=jax.ShapeDtypeStruct((NCORE * NPAD, C), jnp.float32),
    mesh=_mesh,
    compiler_params=_sc_params,
    scratch_types=[
        pltpu.VMEM((SCH, SIW), jnp.int32),
        pltpu.VMEM((SCH, SIW), jnp.int32),
        pltpu.VMEM((SIW, C), jnp.float32),
        pltpu.VMEM((SIW, C), jnp.float32),
        pltpu.VMEM_SHARED((NPAD, C), jnp.float32),
        pltpu.SemaphoreType.DMA,
    ],
)
def _sc_msgpass(x_h, filt_h, src3_h, dst3_h, parts_h,
                sidx, didx, rows, fv, acc, sem):
    cid = lax.axis_index("c")
    sid = lax.axis_index("s")
    wid = cid * NSUB + sid
    zrows = NPAD // NSUB
    ebase = wid * E_PER_W
    ibase = wid * SJROWS

    def zr(r, _):
        for c8 in range(C // 16):
            fv[r, pl.ds(c8 * 16, 16)] = jnp.zeros((16,), jnp.float32)
        return 0
    lax.fori_loop(0, SIW, zr, 0)

    def zc(i, _):
        pltpu.sync_copy(fv, acc.at[pl.ds(sid * zrows + i * SIW, SIW)])
        return 0
    lax.fori_loop(0, zrows // SIW, zc, 0)
    plsc.subcore_barrier()

    def chunk_body(c, _):
        pltpu.sync_copy(src3_h.at[pl.ds(ibase + c * SCH, SCH)], sidx)
        pltpu.sync_copy(dst3_h.at[pl.ds(ibase + c * SCH, SCH)], didx)

        def jb(jj, _):
            j = c * SCH + jj
            off = jnp.minimum(ebase + j * SIW, N_EDGES - SIW)
            pltpu.async_copy(x_h.at[sidx.at[jj]], rows, sem).wait()
            pltpu.sync_copy(filt_h.at[pl.ds(off, SIW)], fv)

            def mb(r, _):
                for u in range(4):
                    for c8 in range(C // 16):
                        sl = pl.ds(c8 * 16, 16)
                        fv[r * 4 + u, sl] = fv[r * 4 + u, sl] * rows[r * 4 + u, sl]
                return 0
            lax.fori_loop(0, SIW // 4, mb, 0)

            pltpu.sync_copy(fv, acc.at[didx.at[jj]], add=True)
            return 0

        lax.fori_loop(0, SCH, jb, 0)
        return 0

    lax.fori_loop(0, SJROWS // SCH, chunk_body, 0)
    plsc.subcore_barrier()

    pltpu.sync_copy(acc.at[pl.ds(sid * zrows, zrows)],
                    parts_h.at[pl.ds(cid * NPAD + sid * zrows, zrows)])


def _sigmoid(z):
    return 1.0 / (1.0 + jnp.exp(-z))


def _filter_body(sq_ref, w_ref, b_ref, f_ref):
    d = jnp.sqrt(sq_ref[:] + 1e-8)
    cent = lax.broadcasted_iota(jnp.int32, (1, NB), 1).astype(jnp.float32)
    cent = cent * (CUTOFF / (NB - 1))
    diff = d - cent
    rbf = jnp.exp(-GAMMA * diff * diff)
    z = jnp.dot(rbf, w_ref[:], preferred_element_type=jnp.float32) + b_ref[:]
    f_ref[:] = z * _sigmoid(z)


def _tc_filter(sq2, w, b):
    grid = N_EDGES // EB
    return pl.pallas_call(
        _filter_body,
        grid=(grid,),
        in_specs=[
            pl.BlockSpec((EB, 1), lambda i: (i, 0)),
            pl.BlockSpec((NB, C), lambda i: (0, 0)),
            pl.BlockSpec((1, C), lambda i: (0, 0)),
        ],
        out_specs=pl.BlockSpec((EB, C), lambda i: (i, 0)),
        out_shape=jax.ShapeDtypeStruct((N_EDGES, C), jnp.float32),
    )(sq2, w, b)


def _update_body(x_ref, p_ref, w_ref, b_ref, o_ref):
    h = x_ref[:] + p_ref[0] + p_ref[1]
    z = jnp.dot(h, w_ref[:], preferred_element_type=jnp.float32) + b_ref[:]
    o_ref[:] = z * _sigmoid(z)


def _tc_update(x, parts, w, b):
    grid = NPAD // NBLK
    return pl.pallas_call(
        _update_body,
        grid=(grid,),
        in_specs=[
            pl.BlockSpec((NBLK, C), lambda i: (i, 0)),
            pl.BlockSpec((NCORE, NBLK, C), lambda i: (0, i, 0)),
            pl.BlockSpec((C, C), lambda i: (0, 0)),
            pl.BlockSpec((1, C), lambda i: (0, 0)),
        ],
        out_specs=pl.BlockSpec((NBLK, C), lambda i: (i, 0)),
        out_shape=jax.ShapeDtypeStruct((NPAD, C), jnp.float32),
    )(x, parts, w, b)


def _final_body(x_ref, p_ref, w_ref, b_ref, bat_ref, wo_ref, bo_ref,
                gx_ref, gc_ref, go_ref):
    i = pl.program_id(0)
    h = x_ref[:] + p_ref[0] + p_ref[1]
    z = jnp.dot(h, w_ref[:], preferred_element_type=jnp.float32) + b_ref[:]
    x2 = z * _sigmoid(z)
    gids = lax.broadcasted_iota(jnp.int32, (1, G), 1)
    onehot = (bat_ref[:] == gids).astype(jnp.float32)
    dn = (((0,), (0,)), ((), ()))
    gpart = lax.dot_general(onehot, x2, dn, preferred_element_type=jnp.float32)
    cpart = lax.dot_general(onehot, jnp.ones_like(x2), dn,
                            preferred_element_type=jnp.float32)

    @pl.when(i == 0)
    def _():
        gx_ref[:] = gpart
        gc_ref[:] = cpart

    @pl.when(i > 0)
    def _():
        gx_ref[:] = gx_ref[:] + gpart
        gc_ref[:] = gc_ref[:] + cpart

    @pl.when(i == pl.num_programs(0) - 1)
    def _():
        go_ref[:] = (jnp.dot(gx_ref[:], wo_ref[:],
                             preferred_element_type=jnp.float32)
                     + gc_ref[:, 0:1] * bo_ref[0, 0])


def _tc_final(x, parts, w, b, bat2, wo, bo):
    grid = NPAD // NBLK
    _, _, go = pl.pallas_call(
        _final_body,
        grid=(grid,),
        in_specs=[
            pl.BlockSpec((NBLK, C), lambda i: (i, 0)),
            pl.BlockSpec((NCORE, NBLK, C), lambda i: (0, i, 0)),
            pl.BlockSpec((C, C), lambda i: (0, 0)),
            pl.BlockSpec((1, C), lambda i: (0, 0)),
            pl.BlockSpec((NBLK, 1), lambda i: (i, 0)),
            pl.BlockSpec((C, 1), lambda i: (0, 0)),
            pl.BlockSpec((1, 1), lambda i: (0, 0)),
        ],
        out_specs=[
            pl.BlockSpec((G, C), lambda i: (0, 0)),
            pl.BlockSpec((G, C), lambda i: (0, 0)),
            pl.BlockSpec((G, 1), lambda i: (0, 0)),
        ],
        out_shape=[
            jax.ShapeDtypeStruct((G, C), jnp.float32),
            jax.ShapeDtypeStruct((G, C), jnp.float32),
            jax.ShapeDtypeStruct((G, 1), jnp.float32),
        ],
        compiler_params=pltpu.CompilerParams(
            dimension_semantics=("arbitrary",)),
    )(x, parts, w, b, bat2, wo, bo)
    return go


def kernel(atomic_number, coordinate, edge_index, batch, embed_table,
           W_rbf0, b_rbf0, W_up0, b_up0,
           W_rbf1, b_rbf1, W_up1, b_up1,
           W_out, b_out):
    an = jnp.pad(atomic_number.astype(jnp.int32), (0, NPAD - N_NODES))
    an2d = an.reshape(NPAD // IW, IW)
    coord_t = jnp.pad(coordinate.T.astype(jnp.float32),
                      ((0, 0), (0, NPAD - N_NODES)))
    cx, cy, cz = coord_t[0], coord_t[1], coord_t[2]
    src = edge_index[0].astype(jnp.int32)
    dst = edge_index[1].astype(jnp.int32)
    src3 = jnp.pad(src.reshape(NW, E_PER_W),
                   ((0, 0), (0, E_TILE - E_PER_W))
                   ).reshape(NW * SJROWS, SIW)
    dst3 = jnp.pad(dst.reshape(NW, E_PER_W),
                   ((0, 0), (0, E_TILE - E_PER_W)),
                   constant_values=TRASH).reshape(NW * SJROWS, SIW)
    bat2 = jnp.pad(batch.astype(jnp.int32), (0, NPAD - N_NODES),
                   constant_values=G).reshape(NPAD, 1)

    x0, sq = _sc_prep(an2d, cx, cy, cz, src, dst, embed_table)
    sq2 = sq.reshape(N_EDGES, 1)
    filt0 = _tc_filter(sq2, W_rbf0, b_rbf0.reshape(1, C))
    filt1 = _tc_filter(sq2, W_rbf1, b_rbf1.reshape(1, C))
    parts0 = _sc_msgpass(x0, filt0, src3, dst3).reshape(NCORE, NPAD, C)
    x1 = _tc_update(x0, parts0, W_up0, b_up0.reshape(1, C))
    parts1 = _sc_msgpass(x1, filt1, src3, dst3).reshape(NCORE, NPAD, C)
    go = _tc_final(x1, parts1, W_up1, b_up1.reshape(1, C),
                   bat2, W_out, b_out.reshape(1, 1))
    return go


# speedup vs baseline: 1.0147x; 1.0033x over previous
import functools

import jax
import jax.numpy as jnp
from jax import lax
from jax.experimental import pallas as pl
from jax.experimental.pallas import tpu as pltpu
from jax.experimental.pallas import tpu_sc as plsc

N_NODES = 10000
N_EDGES = 320000
C = 128
NB = 32
N_SPECIES = 100
G = 64
CUTOFF = 5.0
GAMMA = (NB / CUTOFF) ** 2

NCORE = 2
NSUB = 16
NW = NCORE * NSUB
NPAD = 10240
ROWS_PER_W = NPAD // NW
E_PER_W = N_EDGES // NW
IW = 80
SIW = 128
E_TILE = 10240
HS = 64
SJROWS = E_TILE // HS
SCH = 16
TRASH = NPAD - 1
ECH = 2000
EB = 512
NBLK = 256

_mesh = plsc.VectorSubcoreMesh(
    core_axis_name="c", subcore_axis_name="s",
    num_cores=NCORE, num_subcores=NSUB)
_sc_params = pltpu.CompilerParams(needs_layout_passes=False)


@functools.partial(
    pl.kernel,
    out_type=[jax.ShapeDtypeStruct((NPAD, C), jnp.float32),
              jax.ShapeDtypeStruct((N_EDGES,), jnp.float32)],
    mesh=_mesh,
    compiler_params=_sc_params,
    scratch_types=[
        pltpu.VMEM((NPAD,), jnp.float32),
        pltpu.VMEM((NPAD,), jnp.float32),
        pltpu.VMEM((NPAD,), jnp.float32),
        pltpu.VMEM((ECH,), jnp.int32),
        pltpu.VMEM((ECH,), jnp.int32),
        pltpu.VMEM((ECH,), jnp.float32),
        pltpu.VMEM((4, IW), jnp.int32),
        pltpu.VMEM((IW, C), jnp.float32),
        pltpu.SemaphoreType.DMA,
    ],
)
def _sc_prep(an2d_h, cx_h, cy_h, cz_h, src_h, dst_h, embed_h, x0_h, sq_h,
             cxv, cyv, czv, sidx, didx, sqv, anv, xrows, sem):
    cid = lax.axis_index("c")
    sid = lax.axis_index("s")
    wid = cid * NSUB + sid

    pltpu.sync_copy(an2d_h.at[pl.ds(wid * 4, 4)], anv)
    for j in range(4):
        pltpu.async_copy(embed_h.at[anv.at[j]], xrows, sem).wait()
        pltpu.sync_copy(xrows, x0_h.at[pl.ds(wid * ROWS_PER_W + j * IW, IW)])

    pltpu.sync_copy(cx_h, cxv)
    pltpu.sync_copy(cy_h, cyv)
    pltpu.sync_copy(cz_h, czv)

    ebase = wid * E_PER_W

    def chunk_body(k, _):
        base = ebase + k * ECH
        pltpu.sync_copy(src_h.at[pl.ds(base, ECH)], sidx)
        pltpu.sync_copy(dst_h.at[pl.ds(base, ECH)], didx)

        def g_body(g, _):
            sl = pl.ds(g * 16, 16)
            s16 = sidx[sl]
            d16 = didx[sl]
            dx = plsc.load_gather(cxv, [d16]) - plsc.load_gather(cxv, [s16])
            dy = plsc.load_gather(cyv, [d16]) - plsc.load_gather(cyv, [s16])
            dz = plsc.load_gather(czv, [d16]) - plsc.load_gather(czv, [s16])
            sqv[sl] = dx * dx + dy * dy + dz * dz
            return 0

        lax.fori_loop(0, ECH // 16, g_body, 0)
        pltpu.sync_copy(sqv, sq_h.at[pl.ds(base, ECH)])
        return 0

    lax.fori_loop(0, E_PER_W // ECH, chunk_body, 0)


@functools.partial(
    pl.kernel,
    out_type=jax.ShapeDtypeStruct((NCORE * NPAD, C), jnp.float32),
    mesh=_mesh,
    compiler_params=_sc_params,
    scratch_types=[
        pltpu.VMEM((SCH, HS), jnp.int32),
        pltpu.VMEM((SCH, HS), jnp.int32),
        pltpu.VMEM((HS, C), jnp.float32),
        pltpu.VMEM((HS, C), jnp.float32),
        pltpu.VMEM((HS, C), jnp.float32),
        pltpu.VMEM_SHARED((NPAD, C), jnp.float32),
        pltpu.SemaphoreType.DMA,
        pltpu.SemaphoreType.DMA,
    ],
)
def _sc_msgpass(x_h, filt_h, src3_h, dst3_h, parts_h,
                sidx, didx, rows0, rows1, fv, acc, sem0, sem1):
    cid = lax.axis_index("c")
    sid = lax.axis_index("s")
    wid = cid * NSUB + sid
    zrows = NPAD // NSUB
    ebase = wid * E_PER_W
    ibase = wid * SJROWS

    def zr(r, _):
        for c8 in range(C // 16):
            fv[r, pl.ds(c8 * 16, 16)] = jnp.zeros((16,), jnp.float32)
        return 0
    lax.fori_loop(0, HS, zr, 0)

    def zc(i, _):
        pltpu.sync_copy(fv, acc.at[pl.ds(sid * zrows + i * HS, HS)])
        return 0
    lax.fori_loop(0, zrows // HS, zc, 0)
    plsc.subcore_barrier()

    rbufs = (rows0, rows1)
    sems = (sem0, sem1)

    def chunk_body(c, _):
        pltpu.sync_copy(src3_h.at[pl.ds(ibase + c * SCH, SCH)], sidx)
        pltpu.sync_copy(dst3_h.at[pl.ds(ibase + c * SCH, SCH)], didx)

        handles = [None, None]
        handles[0] = pltpu.async_copy(x_h.at[sidx.at[0]], rows0, sem0)
        for jj in range(SCH):
            if jj + 1 < SCH:
                nb = (jj + 1) % 2
                handles[nb] = pltpu.async_copy(
                    x_h.at[sidx.at[jj + 1]], rbufs[nb], sems[nb])
            j = c * SCH + jj
            off = jnp.minimum(ebase + j * HS, N_EDGES - HS)
            handles[jj % 2].wait()
            pltpu.sync_copy(filt_h.at[pl.ds(off, HS)], fv)
            rcur = rbufs[jj % 2]

            def mb(r, _, rcur=rcur):
                for u in range(4):
                    for c8 in range(C // 16):
                        sl = pl.ds(c8 * 16, 16)
                        fv[r * 4 + u, sl] = fv[r * 4 + u, sl] * rcur[r * 4 + u, sl]
                return 0
            lax.fori_loop(0, HS // 4, mb, 0)

            pltpu.sync_copy(fv, acc.at[didx.at[jj]], add=True)
        return 0

    lax.fori_loop(0, SJROWS // SCH, chunk_body, 0)
    plsc.subcore_barrier()

    pltpu.sync_copy(acc.at[pl.ds(sid * zrows, zrows)],
                    parts_h.at[pl.ds(cid * NPAD + sid * zrows, zrows)])


def _sigmoid(z):
    return 1.0 / (1.0 + jnp.exp(-z))


def _filter_body(sq_ref, w_ref, b_ref, f_ref):
    d = jnp.sqrt(sq_ref[:] + 1e-8)
    cent = lax.broadcasted_iota(jnp.int32, (1, NB), 1).astype(jnp.float32)
    cent = cent * (CUTOFF / (NB - 1))
    diff = d - cent
    rbf = jnp.exp(-GAMMA * diff * diff)
    z = jnp.dot(rbf, w_ref[:], preferred_element_type=jnp.float32) + b_ref[:]
    f_ref[:] = z * _sigmoid(z)


def _tc_filter(sq2, w, b):
    grid = N_EDGES // EB
    return pl.pallas_call(
        _filter_body,
        grid=(grid,),
        in_specs=[
            pl.BlockSpec((EB, 1), lambda i: (i, 0)),
            pl.BlockSpec((NB, C), lambda i: (0, 0)),
            pl.BlockSpec((1, C), lambda i: (0, 0)),
        ],
        out_specs=pl.BlockSpec((EB, C), lambda i: (i, 0)),
        out_shape=jax.ShapeDtypeStruct((N_EDGES, C), jnp.float32),
    )(sq2, w, b)


def _update_body(x_ref, p_ref, w_ref, b_ref, o_ref):
    h = x_ref[:] + p_ref[0] + p_ref[1]
    z = jnp.dot(h, w_ref[:], preferred_element_type=jnp.float32) + b_ref[:]
    o_ref[:] = z * _sigmoid(z)


def _tc_update(x, parts, w, b):
    grid = NPAD // NBLK
    return pl.pallas_call(
        _update_body,
        grid=(grid,),
        in_specs=[
            pl.BlockSpec((NBLK, C), lambda i: (i, 0)),
            pl.BlockSpec((NCORE, NBLK, C), lambda i: (0, i, 0)),
            pl.BlockSpec((C, C), lambda i: (0, 0)),
            pl.BlockSpec((1, C), lambda i: (0, 0)),
        ],
        out_specs=pl.BlockSpec((NBLK, C), lambda i: (i, 0)),
        out_shape=jax.ShapeDtypeStruct((NPAD, C), jnp.float32),
    )(x, parts, w, b)


def _final_body(x_ref, p_ref, w_ref, b_ref, bat_ref, wo_ref, bo_ref,
                gx_ref, gc_ref, go_ref):
    i = pl.program_id(0)
    h = x_ref[:] + p_ref[0] + p_ref[1]
    z = jnp.dot(h, w_ref[:], preferred_element_type=jnp.float32) + b_ref[:]
    x2 = z * _sigmoid(z)
    gids = lax.broadcasted_iota(jnp.int32, (1, G), 1)
    onehot = (bat_ref[:] == gids).astype(jnp.float32)
    dn = (((0,), (0,)), ((), ()))
    gpart = lax.dot_general(onehot, x2, dn, preferred_element_type=jnp.float32)
    cpart = lax.dot_general(onehot, jnp.ones_like(x2), dn,
                            preferred_element_type=jnp.float32)

    @pl.when(i == 0)
    def _():
        gx_ref[:] = gpart
        gc_ref[:] = cpart

    @pl.when(i > 0)
    def _():
        gx_ref[:] = gx_ref[:] + gpart
        gc_ref[:] = gc_ref[:] + cpart

    @pl.when(i == pl.num_programs(0) - 1)
    def _():
        go_ref[:] = (jnp.dot(gx_ref[:], wo_ref[:],
                             preferred_element_type=jnp.float32)
                     + gc_ref[:, 0:1] * bo_ref[0, 0])


def _tc_final(x, parts, w, b, bat2, wo, bo):
    grid = NPAD // NBLK
    _, _, go = pl.pallas_call(
        _final_body,
        grid=(grid,),
        in_specs=[
            pl.BlockSpec((NBLK, C), lambda i: (i, 0)),
            pl.BlockSpec((NCORE, NBLK, C), lambda i: (0, i, 0)),
            pl.BlockSpec((C, C), lambda i: (0, 0)),
            pl.BlockSpec((1, C), lambda i: (0, 0)),
            pl.BlockSpec((NBLK, 1), lambda i: (i, 0)),
            pl.BlockSpec((C, 1), lambda i: (0, 0)),
            pl.BlockSpec((1, 1), lambda i: (0, 0)),
        ],
        out_specs=[
            pl.BlockSpec((G, C), lambda i: (0, 0)),
            pl.BlockSpec((G, C), lambda i: (0, 0)),
            pl.BlockSpec((G, 1), lambda i: (0, 0)),
        ],
        out_shape=[
            jax.ShapeDtypeStruct((G, C), jnp.float32),
            jax.ShapeDtypeStruct((G, C), jnp.float32),
            jax.ShapeDtypeStruct((G, 1), jnp.float32),
        ],
        compiler_params=pltpu.CompilerParams(
            dimension_semantics=("arbitrary",)),
    )(x, parts, w, b, bat2, wo, bo)
    return go


def kernel(atomic_number, coordinate, edge_index, batch, embed_table,
           W_rbf0, b_rbf0, W_up0, b_up0,
           W_rbf1, b_rbf1, W_up1, b_up1,
           W_out, b_out):
    an = jnp.pad(atomic_number.astype(jnp.int32), (0, NPAD - N_NODES))
    an2d = an.reshape(NPAD // IW, IW)
    coord_t = jnp.pad(coordinate.T.astype(jnp.float32),
                      ((0, 0), (0, NPAD - N_NODES)))
    cx, cy, cz = coord_t[0], coord_t[1], coord_t[2]
    src = edge_index[0].astype(jnp.int32)
    dst = edge_index[1].astype(jnp.int32)
    src3 = jnp.pad(src.reshape(NW, E_PER_W),
                   ((0, 0), (0, E_TILE - E_PER_W))
                   ).reshape(NW * SJROWS, HS)
    dst3 = jnp.pad(dst.reshape(NW, E_PER_W),
                   ((0, 0), (0, E_TILE - E_PER_W)),
                   constant_values=TRASH).reshape(NW * SJROWS, HS)
    bat2 = jnp.pad(batch.astype(jnp.int32), (0, NPAD - N_NODES),
                   constant_values=G).reshape(NPAD, 1)

    x0, sq = _sc_prep(an2d, cx, cy, cz, src, dst, embed_table)
    sq2 = sq.reshape(N_EDGES, 1)
    filt0 = _tc_filter(sq2, W_rbf0, b_rbf0.reshape(1, C))
    filt1 = _tc_filter(sq2, W_rbf1, b_rbf1.reshape(1, C))
    parts0 = _sc_msgpass(x0, filt0, src3, dst3).reshape(NCORE, NPAD, C)
    x1 = _tc_update(x0, parts0, W_up0, b_up0.reshape(1, C))
    parts1 = _sc_msgpass(x1, filt1, src3, dst3).reshape(NCORE, NPAD, C)
    go = _tc_final(x1, parts1, W_up1, b_up1.reshape(1, C),
                   bat2, W_out, b_out.reshape(1, 1))
    return go


# speedup vs baseline: 1.1357x; 1.1192x over previous
import functools

import jax
import jax.numpy as jnp
from jax import lax
from jax.experimental import pallas as pl
from jax.experimental.pallas import tpu as pltpu
from jax.experimental.pallas import tpu_sc as plsc

N_NODES = 10000
N_EDGES = 320000
C = 128
NB = 32
N_SPECIES = 100
G = 64
CUTOFF = 5.0
GAMMA = (NB / CUTOFF) ** 2

NCORE = 2
NSUB = 16
NW = NCORE * NSUB
NPAD = 10240
ROWS_PER_W = NPAD // NW
E_PER_W = N_EDGES // NW
IW = 80
SIW = 128
E_TILE = 10240
HS = 64
SJROWS = E_TILE // HS
SCH = 16
TRASH = NPAD - 1
ECH = 2000
EB = 512
NBLK = 256

_mesh = plsc.VectorSubcoreMesh(
    core_axis_name="c", subcore_axis_name="s",
    num_cores=NCORE, num_subcores=NSUB)
_sc_params = pltpu.CompilerParams(needs_layout_passes=False)


@functools.partial(
    pl.kernel,
    out_type=[jax.ShapeDtypeStruct((NPAD, C), jnp.float32),
              jax.ShapeDtypeStruct((N_EDGES,), jnp.float32)],
    mesh=_mesh,
    compiler_params=_sc_params,
    scratch_types=[
        pltpu.VMEM((NPAD,), jnp.float32),
        pltpu.VMEM((NPAD,), jnp.float32),
        pltpu.VMEM((NPAD,), jnp.float32),
        pltpu.VMEM((ECH,), jnp.int32),
        pltpu.VMEM((ECH,), jnp.int32),
        pltpu.VMEM((ECH,), jnp.float32),
        pltpu.VMEM((4, IW), jnp.int32),
        pltpu.VMEM((IW, C), jnp.float32),
        pltpu.SemaphoreType.DMA,
    ],
)
def _sc_prep(an2d_h, cx_h, cy_h, cz_h, src_h, dst_h, embed_h, x0_h, sq_h,
             cxv, cyv, czv, sidx, didx, sqv, anv, xrows, sem):
    cid = lax.axis_index("c")
    sid = lax.axis_index("s")
    wid = cid * NSUB + sid

    pltpu.sync_copy(an2d_h.at[pl.ds(wid * 4, 4)], anv)
    for j in range(4):
        pltpu.async_copy(embed_h.at[anv.at[j]], xrows, sem).wait()
        pltpu.sync_copy(xrows, x0_h.at[pl.ds(wid * ROWS_PER_W + j * IW, IW)])

    pltpu.sync_copy(cx_h, cxv)
    pltpu.sync_copy(cy_h, cyv)
    pltpu.sync_copy(cz_h, czv)

    ebase = wid * E_PER_W

    def chunk_body(k, _):
        base = ebase + k * ECH
        pltpu.sync_copy(src_h.at[pl.ds(base, ECH)], sidx)
        pltpu.sync_copy(dst_h.at[pl.ds(base, ECH)], didx)

        def g_body(g, _):
            sl = pl.ds(g * 16, 16)
            s16 = sidx[sl]
            d16 = didx[sl]
            dx = plsc.load_gather(cxv, [d16]) - plsc.load_gather(cxv, [s16])
            dy = plsc.load_gather(cyv, [d16]) - plsc.load_gather(cyv, [s16])
            dz = plsc.load_gather(czv, [d16]) - plsc.load_gather(czv, [s16])
            sqv[sl] = dx * dx + dy * dy + dz * dz
            return 0

        lax.fori_loop(0, ECH // 16, g_body, 0)
        pltpu.sync_copy(sqv, sq_h.at[pl.ds(base, ECH)])
        return 0

    lax.fori_loop(0, E_PER_W // ECH, chunk_body, 0)


@functools.partial(
    pl.kernel,
    out_type=jax.ShapeDtypeStruct((NCORE * NPAD, C), jnp.float32),
    mesh=_mesh,
    compiler_params=_sc_params,
    scratch_types=[
        pltpu.VMEM((SCH, HS), jnp.int32),
        pltpu.VMEM((SCH, HS), jnp.int32),
        pltpu.VMEM((HS, C), jnp.float32),
        pltpu.VMEM((HS, C), jnp.float32),
        pltpu.VMEM((HS, C), jnp.float32),
        pltpu.VMEM((HS, C), jnp.float32),
        pltpu.VMEM_SHARED((NPAD, C), jnp.float32),
        pltpu.SemaphoreType.DMA,
        pltpu.SemaphoreType.DMA,
        pltpu.SemaphoreType.DMA,
        pltpu.SemaphoreType.DMA,
    ],
)
def _sc_msgpass(x_h, filt_h, src3_h, dst3_h, parts_h,
                sidx, didx, rows0, rows1, fv0, fv1, acc,
                gsem0, gsem1, fsem0, fsem1):
    cid = lax.axis_index("c")
    sid = lax.axis_index("s")
    wid = cid * NSUB + sid
    zrows = NPAD // NSUB
    ebase = wid * E_PER_W
    ibase = wid * SJROWS

    def zr(r, _):
        for c8 in range(C // 16):
            fv0[r, pl.ds(c8 * 16, 16)] = jnp.zeros((16,), jnp.float32)
        return 0
    lax.fori_loop(0, HS, zr, 0)

    def zc(i, _):
        pltpu.sync_copy(fv0, acc.at[pl.ds(sid * zrows + i * HS, HS)])
        return 0
    lax.fori_loop(0, zrows // HS, zc, 0)
    plsc.subcore_barrier()

    rbufs = (rows0, rows1)
    fbufs = (fv0, fv1)
    gsems = (gsem0, gsem1)
    fsems = (fsem0, fsem1)

    def foff(c, jj):
        return jnp.minimum(ebase + (c * SCH + jj) * HS, N_EDGES - HS)

    def chunk_body(c, _):
        pltpu.sync_copy(src3_h.at[pl.ds(ibase + c * SCH, SCH)], sidx)
        pltpu.sync_copy(dst3_h.at[pl.ds(ibase + c * SCH, SCH)], didx)

        gh = [None, None]
        fh = [None, None]
        gh[0] = pltpu.async_copy(x_h.at[sidx.at[0]], rows0, gsem0)
        fh[0] = pltpu.async_copy(filt_h.at[pl.ds(foff(c, 0), HS)], fv0, fsem0)
        for jj in range(SCH):
            cur = jj % 2
            if jj + 1 < SCH:
                nb = (jj + 1) % 2
                gh[nb] = pltpu.async_copy(
                    x_h.at[sidx.at[jj + 1]], rbufs[nb], gsems[nb])
                fh[nb] = pltpu.async_copy(
                    filt_h.at[pl.ds(foff(c, jj + 1), HS)], fbufs[nb], fsems[nb])
            gh[cur].wait()
            fh[cur].wait()
            rcur = rbufs[cur]
            fcur = fbufs[cur]

            def mb(r, _, rcur=rcur, fcur=fcur):
                for u in range(4):
                    for c8 in range(C // 16):
                        sl = pl.ds(c8 * 16, 16)
                        fcur[r * 4 + u, sl] = fcur[r * 4 + u, sl] * rcur[r * 4 + u, sl]
                return 0
            lax.fori_loop(0, HS // 4, mb, 0)

            pltpu.sync_copy(fcur, acc.at[didx.at[jj]], add=True)
        return 0

    lax.fori_loop(0, SJROWS // SCH, chunk_body, 0)
    plsc.subcore_barrier()

    pltpu.sync_copy(acc.at[pl.ds(sid * zrows, zrows)],
                    parts_h.at[pl.ds(cid * NPAD + sid * zrows, zrows)])


def _sigmoid(z):
    return 1.0 / (1.0 + jnp.exp(-z))


def _filter_body(sq_ref, w_ref, b_ref, f_ref):
    d = jnp.sqrt(sq_ref[:] + 1e-8)
    cent = lax.broadcasted_iota(jnp.int32, (1, NB), 1).astype(jnp.float32)
    cent = cent * (CUTOFF / (NB - 1))
    diff = d - cent
    rbf = jnp.exp(-GAMMA * diff * diff)
    z = jnp.dot(rbf, w_ref[:], preferred_element_type=jnp.float32) + b_ref[:]
    f_ref[:] = z * _sigmoid(z)


def _tc_filter(sq2, w, b):
    grid = N_EDGES // EB
    return pl.pallas_call(
        _filter_body,
        grid=(grid,),
        in_specs=[
            pl.BlockSpec((EB, 1), lambda i: (i, 0)),
            pl.BlockSpec((NB, C), lambda i: (0, 0)),
            pl.BlockSpec((1, C), lambda i: (0, 0)),
        ],
        out_specs=pl.BlockSpec((EB, C), lambda i: (i, 0)),
        out_shape=jax.ShapeDtypeStruct((N_EDGES, C), jnp.float32),
    )(sq2, w, b)


def _update_body(x_ref, p_ref, w_ref, b_ref, o_ref):
    h = x_ref[:] + p_ref[0] + p_ref[1]
    z = jnp.dot(h, w_ref[:], preferred_element_type=jnp.float32) + b_ref[:]
    o_ref[:] = z * _sigmoid(z)


def _tc_update(x, parts, w, b):
    grid = NPAD // NBLK
    return pl.pallas_call(
        _update_body,
        grid=(grid,),
        in_specs=[
            pl.BlockSpec((NBLK, C), lambda i: (i, 0)),
            pl.BlockSpec((NCORE, NBLK, C), lambda i: (0, i, 0)),
            pl.BlockSpec((C, C), lambda i: (0, 0)),
            pl.BlockSpec((1, C), lambda i: (0, 0)),
        ],
        out_specs=pl.BlockSpec((NBLK, C), lambda i: (i, 0)),
        out_shape=jax.ShapeDtypeStruct((NPAD, C), jnp.float32),
    )(x, parts, w, b)


def _final_body(x_ref, p_ref, w_ref, b_ref, bat_ref, wo_ref, bo_ref,
                gx_ref, gc_ref, go_ref):
    i = pl.program_id(0)
    h = x_ref[:] + p_ref[0] + p_ref[1]
    z = jnp.dot(h, w_ref[:], preferred_element_type=jnp.float32) + b_ref[:]
    x2 = z * _sigmoid(z)
    gids = lax.broadcasted_iota(jnp.int32, (1, G), 1)
    onehot = (bat_ref[:] == gids).astype(jnp.float32)
    dn = (((0,), (0,)), ((), ()))
    gpart = lax.dot_general(onehot, x2, dn, preferred_element_type=jnp.float32)
    cpart = lax.dot_general(onehot, jnp.ones_like(x2), dn,
                            preferred_element_type=jnp.float32)

    @pl.when(i == 0)
    def _():
        gx_ref[:] = gpart
        gc_ref[:] = cpart

    @pl.when(i > 0)
    def _():
        gx_ref[:] = gx_ref[:] + gpart
        gc_ref[:] = gc_ref[:] + cpart

    @pl.when(i == pl.num_programs(0) - 1)
    def _():
        go_ref[:] = (jnp.dot(gx_ref[:], wo_ref[:],
                             preferred_element_type=jnp.float32)
                     + gc_ref[:, 0:1] * bo_ref[0, 0])


def _tc_final(x, parts, w, b, bat2, wo, bo):
    grid = NPAD // NBLK
    _, _, go = pl.pallas_call(
        _final_body,
        grid=(grid,),
        in_specs=[
            pl.BlockSpec((NBLK, C), lambda i: (i, 0)),
            pl.BlockSpec((NCORE, NBLK, C), lambda i: (0, i, 0)),
            pl.BlockSpec((C, C), lambda i: (0, 0)),
            pl.BlockSpec((1, C), lambda i: (0, 0)),
            pl.BlockSpec((NBLK, 1), lambda i: (i, 0)),
            pl.BlockSpec((C, 1), lambda i: (0, 0)),
            pl.BlockSpec((1, 1), lambda i: (0, 0)),
        ],
        out_specs=[
            pl.BlockSpec((G, C), lambda i: (0, 0)),
            pl.BlockSpec((G, C), lambda i: (0, 0)),
            pl.BlockSpec((G, 1), lambda i: (0, 0)),
        ],
        out_shape=[
            jax.ShapeDtypeStruct((G, C), jnp.float32),
            jax.ShapeDtypeStruct((G, C), jnp.float32),
            jax.ShapeDtypeStruct((G, 1), jnp.float32),
        ],
        compiler_params=pltpu.CompilerParams(
            dimension_semantics=("arbitrary",)),
    )(x, parts, w, b, bat2, wo, bo)
    return go


def kernel(atomic_number, coordinate, edge_index, batch, embed_table,
           W_rbf0, b_rbf0, W_up0, b_up0,
           W_rbf1, b_rbf1, W_up1, b_up1,
           W_out, b_out):
    an = jnp.pad(atomic_number.astype(jnp.int32), (0, NPAD - N_NODES))
    an2d = an.reshape(NPAD // IW, IW)
    coord_t = jnp.pad(coordinate.T.astype(jnp.float32),
                      ((0, 0), (0, NPAD - N_NODES)))
    cx, cy, cz = coord_t[0], coord_t[1], coord_t[2]
    src = edge_index[0].astype(jnp.int32)
    dst = edge_index[1].astype(jnp.int32)
    src3 = jnp.pad(src.reshape(NW, E_PER_W),
                   ((0, 0), (0, E_TILE - E_PER_W))
                   ).reshape(NW * SJROWS, HS)
    dst3 = jnp.pad(dst.reshape(NW, E_PER_W),
                   ((0, 0), (0, E_TILE - E_PER_W)),
                   constant_values=TRASH).reshape(NW * SJROWS, HS)
    bat2 = jnp.pad(batch.astype(jnp.int32), (0, NPAD - N_NODES),
                   constant_values=G).reshape(NPAD, 1)

    x0, sq = _sc_prep(an2d, cx, cy, cz, src, dst, embed_table)
    sq2 = sq.reshape(N_EDGES, 1)
    filt0 = _tc_filter(sq2, W_rbf0, b_rbf0.reshape(1, C))
    filt1 = _tc_filter(sq2, W_rbf1, b_rbf1.reshape(1, C))
    parts0 = _sc_msgpass(x0, filt0, src3, dst3).reshape(NCORE, NPAD, C)
    x1 = _tc_update(x0, parts0, W_up0, b_up0.reshape(1, C))
    parts1 = _sc_msgpass(x1, filt1, src3, dst3).reshape(NCORE, NPAD, C)
    go = _tc_final(x1, parts1, W_up1, b_up1.reshape(1, C),
                   bat2, W_out, b_out.reshape(1, 1))
    return go
